# Initial kernel scaffold; baseline (speedup 1.0000x reference)
import jax
import jax.numpy as jnp
from jax.experimental import pallas as pl


def kernel(x, edge_index, W_src, b_src, W_dst, b_dst, attn_a, gamma, beta):
    raise NotImplementedError("write your pallas kernel here")


# speedup vs baseline: 1.0379x; 1.0379x over previous
import functools

import jax
import jax.numpy as jnp
from jax.experimental import pallas as pl
from jax.experimental.pallas import tpu as pltpu

N = 10000
E = 320000
D = 128
H = 8
DH = 16

_ROWS = 1000


def _mm_body(x_ref, ws_ref, bs_ref, wd_ref, bd_ref, fs_ref, fd_ref):
    x = x_ref[...]
    fs_ref[...] = jnp.dot(x, ws_ref[...], preferred_element_type=jnp.float32) + bs_ref[...]
    fd_ref[...] = jnp.dot(x, wd_ref[...], preferred_element_type=jnp.float32) + bd_ref[...]


def _feats(x, W_src, b_src, W_dst, b_dst):
    grid = N // _ROWS
    return pl.pallas_call(
        _mm_body,
        grid=(grid,),
        in_specs=[
            pl.BlockSpec((_ROWS, D), lambda i: (i, 0)),
            pl.BlockSpec((D, D), lambda i: (0, 0)),
            pl.BlockSpec((1, D), lambda i: (0, 0)),
            pl.BlockSpec((D, D), lambda i: (0, 0)),
            pl.BlockSpec((1, D), lambda i: (0, 0)),
        ],
        out_specs=[
            pl.BlockSpec((_ROWS, D), lambda i: (i, 0)),
            pl.BlockSpec((_ROWS, D), lambda i: (i, 0)),
        ],
        out_shape=[
            jax.ShapeDtypeStruct((N, D), jnp.float32),
            jax.ShapeDtypeStruct((N, D), jnp.float32),
        ],
    )(x, W_src, b_src.reshape(1, D), W_dst, b_dst.reshape(1, D))


def _bn_body(msg_ref, x_ref, g_ref, b_ref, out_ref):
    feat = msg_ref[...] + x_ref[...]
    mean = jnp.mean(feat, axis=0, keepdims=True)
    var = jnp.mean((feat - mean) ** 2, axis=0, keepdims=True)
    y = (feat - mean) * jax.lax.rsqrt(var + 1e-5) * g_ref[...] + b_ref[...]
    out_ref[...] = jnp.maximum(y, 0.0)


def _bn_relu(msg, x, gamma, beta):
    return pl.pallas_call(
        _bn_body,
        out_shape=jax.ShapeDtypeStruct((N, D), jnp.float32),
    )(msg, x, gamma.reshape(1, D), beta.reshape(1, D))


def kernel(x, edge_index, W_src, b_src, W_dst, b_dst, attn_a, gamma, beta):
    src = edge_index[0]
    dst = edge_index[1]
    feat_src, feat_dst = _feats(x, W_src, b_src, W_dst, b_dst)
    fs = feat_src.reshape(N, H, DH)
    e = jax.nn.leaky_relu(feat_src[src].reshape(E, H, DH)
                          + feat_dst[dst].reshape(E, H, DH), negative_slope=0.2)
    logits = jnp.einsum('ehd,hd->eh', e, attn_a)
    m = jax.ops.segment_max(logits, dst, num_segments=N)
    m = jnp.where(jnp.isfinite(m), m, 0.0)
    ex = jnp.exp(logits - m[dst])
    denom = jax.ops.segment_sum(ex, dst, num_segments=N)
    alpha = ex / (denom[dst] + 1e-9)
    msg = jax.ops.segment_sum(fs[src] * alpha[..., None], dst, num_segments=N)
    feat = _bn_relu(msg.reshape(N, D), x, gamma, beta)
    return feat, alpha[..., None]
